# baseline (speedup 1.0000x reference)
import functools

import jax
import jax.numpy as jnp
from jax import lax
from jax.experimental import pallas as pl
from jax.experimental.pallas import tpu as pltpu, tpu_sc as plsc

VOCAB = 1000000
EMBED_DIM = 32
BATCH = 16384
HIST = 200

_NC, _NS = 2, 16
_NW = _NC * _NS
_N = BATCH * HIST
_PER_W = _N // _NW
_NBUF = 4
_CHUNK = 800
_STEPS = _PER_W // _CHUNK
_GROUPS = _STEPS // _NBUF


def _sc_gather(idx_flat, table):
    mesh = plsc.VectorSubcoreMesh(core_axis_name="c", subcore_axis_name="s")

    @functools.partial(
        pl.kernel,
        mesh=mesh,
        out_type=jax.ShapeDtypeStruct((_N, EMBED_DIM), jnp.float32),
        scratch_types=[
            [pltpu.VMEM((_CHUNK,), jnp.int32) for _ in range(_NBUF)],
            [pltpu.VMEM((_CHUNK, EMBED_DIM), jnp.float32) for _ in range(_NBUF)],
            [pltpu.SemaphoreType.DMA for _ in range(_NBUF)],
            [pltpu.SemaphoreType.DMA for _ in range(_NBUF)],
        ],
        compiler_params=pltpu.CompilerParams(use_tc_tiling_on_sc=False),
    )
    def k(idx_hbm, table_hbm, out_hbm, idx_v, rows_v, gsem, osem):
        wid = lax.axis_index("s") * _NC + lax.axis_index("c")
        base = wid * _PER_W

        def launch(chunk, slot):
            off = base + chunk * _CHUNK
            pltpu.sync_copy(idx_hbm.at[pl.ds(off, _CHUNK)], idx_v[slot])
            pltpu.async_copy(table_hbm.at[idx_v[slot]], rows_v[slot], gsem[slot])

        for b in range(_NBUF - 1):
            launch(b, b)

        def body(i2, carry):
            for b in range(_NBUF):
                i = i2 * _NBUF + b
                lb = (b + _NBUF - 1) % _NBUF

                @pl.when(i + _NBUF - 1 < _STEPS)
                def _():
                    @pl.when(i >= 1)
                    def _():
                        pltpu.make_async_copy(
                            rows_v[lb], out_hbm.at[pl.ds(base, _CHUNK)],
                            osem[lb]).wait()
                    launch(i + _NBUF - 1, lb)

                off = base + i * _CHUNK
                pltpu.make_async_copy(
                    table_hbm.at[idx_v[b]], rows_v[b], gsem[b]).wait()
                pltpu.async_copy(rows_v[b], out_hbm.at[pl.ds(off, _CHUNK)],
                                 osem[b])
            return carry

        lax.fori_loop(0, _GROUPS, body, 0)

        for b in range(_NBUF):
            pltpu.make_async_copy(
                rows_v[b], out_hbm.at[pl.ds(base, _CHUNK)], osem[b]).wait()

    return k(idx_flat, table)


def kernel(token_ids, table):
    idx_flat = token_ids.reshape(-1).astype(jnp.int32)
    out = _sc_gather(idx_flat, table)
    return out.reshape(token_ids.shape + (table.shape[1],))


# speedup vs baseline: 1.1288x; 1.1288x over previous
import functools

import jax
import jax.numpy as jnp
from jax import lax
from jax.experimental import pallas as pl
from jax.experimental.pallas import tpu as pltpu, tpu_sc as plsc

VOCAB = 1000000
EMBED_DIM = 32
BATCH = 16384
HIST = 200

_NC, _NS = 2, 16
_NW = _NC * _NS
_BPT = BATCH // _NW
_MAIN = 999936
_CHUNK_A = 62464
_CHUNK_B = _MAIN - 15 * _CHUNK_A
_TAIL0 = VOCAB - 128
_BC2 = 2048
_NB2 = BATCH // _BC2
_J_PER_TILE = HIST * _NB2 // _NW


def _sc_gather(idx_flat, tab_t, tail_t):
    mesh = plsc.VectorSubcoreMesh(core_axis_name="c", subcore_axis_name="s")

    @functools.partial(
        pl.kernel,
        mesh=mesh,
        out_type=(
            jax.ShapeDtypeStruct((HIST, EMBED_DIM, BATCH), jnp.float32),
            jax.ShapeDtypeStruct((HIST * EMBED_DIM * BATCH,), jnp.float32),
        ),
        scratch_types=[
            [pltpu.VMEM((_BPT,), jnp.int32) for _ in range(4)],
            [pltpu.VMEM((_BPT,), jnp.float32) for _ in range(4)],
            [pltpu.VMEM((8, _BC2), jnp.float32) for _ in range(2)],
            pltpu.VMEM((128,), jnp.float32),
            pltpu.VMEM_SHARED((VOCAB,), jnp.float32),
            pltpu.SemaphoreType.DMA,
            [pltpu.SemaphoreType.DMA for _ in range(4)],
            [pltpu.SemaphoreType.DMA for _ in range(4)],
            [pltpu.SemaphoreType.DMA for _ in range(4)],
            [pltpu.SemaphoreType.DMA for _ in range(2)],
            [pltpu.SemaphoreType.DMA for _ in range(2)],
        ],
    )
    def k(idx_hbm, tab_hbm, tail_hbm, out_hbm, sc_hbm, ibuf, gbuf, obuf,
          tbuf, plane, lsem, ism, gsem, osem, rsem, wsem):
        cid = lax.axis_index("c")
        sid = lax.axis_index("s")
        wid = cid * _NS + sid
        b0 = wid * _BPT

        def ifire(h, s):
            pltpu.async_copy(idx_hbm.at[pl.ds(h * BATCH + b0, _BPT)],
                             ibuf[s], ism[s])

        def iwait(h, s):
            pltpu.make_async_copy(idx_hbm.at[pl.ds(h * BATCH + b0, _BPT)],
                                  ibuf[s], ism[s]).wait()

        def sc_pos(e, h):
            return (e * HIST + h) * BATCH

        def eloop(e, carry):
            @pl.when(sid < 15)
            def _():
                pltpu.async_copy(
                    tab_hbm.at[e, pl.ds(sid * _CHUNK_A, _CHUNK_A)],
                    plane.at[pl.ds(sid * _CHUNK_A, _CHUNK_A)], lsem).wait()

            @pl.when(sid == 15)
            def _():
                pltpu.async_copy(
                    tab_hbm.at[e, pl.ds(15 * _CHUNK_A, _CHUNK_B)],
                    plane.at[pl.ds(15 * _CHUNK_A, _CHUNK_B)], lsem).wait()
                pltpu.sync_copy(tail_hbm.at[pl.ds(e * 128, 128)], tbuf)
                pltpu.sync_copy(tbuf, plane.at[pl.ds(_TAIL0, 128)])

            for s in range(4):
                ifire(s, s)
            plsc.subcore_barrier()

            def hloop(h2, carry2):
                for s in range(4):
                    h = h2 * 4 + s
                    iwait(h, s)

                    @pl.when(h2 > 0)
                    def _():
                        pltpu.make_async_copy(
                            gbuf[s], sc_hbm.at[pl.ds(0, _BPT)],
                            osem[s]).wait()

                    pltpu.async_copy(plane.at[ibuf[s]], gbuf[s],
                                     gsem[s]).wait()
                    pltpu.async_copy(
                        gbuf[s],
                        sc_hbm.at[pl.ds(sc_pos(e, h) + b0, _BPT)],
                        osem[s])

                    @pl.when(h + 4 < HIST)
                    def _():
                        ifire(h + 4, s)
                return carry2

            lax.fori_loop(0, HIST // 4, hloop, 0)

            for s in range(4):
                pltpu.make_async_copy(
                    gbuf[s], sc_hbm.at[pl.ds(0, _BPT)], osem[s]).wait()
            plsc.subcore_barrier()

            @pl.when(e % 8 == 7)
            def _():
                eg = e // 8

                def rfire(j, p):
                    h = j // _NB2
                    bc = (j % _NB2) * _BC2
                    for kk in range(8):
                        pltpu.async_copy(
                            sc_hbm.at[
                                pl.ds(sc_pos(eg * 8 + kk, h) + bc, _BC2)],
                            obuf[p].at[kk], rsem[p])

                def rdrain(p):
                    for kk in range(8):
                        pltpu.make_async_copy(
                            sc_hbm.at[pl.ds(0, _BC2)], obuf[p].at[kk],
                            rsem[p]).wait()

                def wfire(j, p):
                    h = j // _NB2
                    bc = (j % _NB2) * _BC2
                    pltpu.async_copy(
                        obuf[p],
                        out_hbm.at[h, pl.ds(eg * 8, 8), pl.ds(bc, _BC2)],
                        wsem[p])

                def wdrain(p):
                    pltpu.make_async_copy(
                        obuf[p], out_hbm.at[0, pl.ds(0, 8), pl.ds(0, _BC2)],
                        wsem[p]).wait()

                j0 = wid * _J_PER_TILE
                rfire(j0, 0)

                def jloop(j2, carry2):
                    for p in range(2):
                        j = j0 + j2 * 2 + p
                        if p == 0:
                            @pl.when(j2 >= 1)
                            def _():
                                wdrain(1)
                            rfire(j + 1, 1)
                        else:
                            @pl.when(j2 < _J_PER_TILE // 2 - 1)
                            def _():
                                wdrain(0)
                                rfire(j + 1, 0)
                        rdrain(p)
                        wfire(j, p)
                    return carry2

                lax.fori_loop(0, _J_PER_TILE // 2, jloop, 0)
                wdrain(0)
                wdrain(1)

            plsc.subcore_barrier()
            return carry

        lax.fori_loop(0, EMBED_DIM, eloop, 0)

    return k(idx_flat, tab_t, tail_t)


def kernel(token_ids, table):
    idx_flat = token_ids.T.reshape(-1).astype(jnp.int32)
    tab_t = table.T
    tail_t = table.T[:, VOCAB - 128:].reshape(-1)
    out3, _ = _sc_gather(idx_flat, tab_t, tail_t)
    return out3.transpose(2, 0, 1)
